# baseline (speedup 1.0000x reference)
import jax
import jax.numpy as jnp
from jax import lax
from jax.experimental import pallas as pl
from jax.experimental.pallas import tpu as pltpu
from jax.experimental.pallas import tpu_sc as plsc

_BATCH = 16384
_PERIOD = 50
_HID = 16

_NC = 2
_NS = 16
_NW = _NC * _NS
_ROWS = _BATCH // _NW
_CH = 64
_NCHUNK = _ROWS // _CH
_ROW_OUT = _PERIOD * _HID
_CH_OUT = _CH * _ROW_OUT


def _body(w_hbm, out_hbm, w_v, buf0, buf1, sem0, sem1):
    wid = lax.axis_index("s") * _NC + lax.axis_index("c")
    base = wid * _ROWS
    pltpu.sync_copy(w_hbm.at[pl.ds(base * _HID, _ROWS * _HID)], w_v)

    bufs = (buf0, buf1)
    sems = (sem0, sem1)

    def fill(buf, ci):
        def row_body(r, carry):
            row = w_v[pl.ds((ci * _CH + r) * _HID, _HID)]
            for p in range(_PERIOD):
                buf[pl.ds(r * _ROW_OUT + p * _HID, _HID)] = row
            return carry
        lax.fori_loop(0, _CH, row_body, 0)

    def dst(ci):
        return out_hbm.at[pl.ds((base + ci * _CH) * _ROW_OUT, _CH_OUT)]

    for b in range(2):
        fill(bufs[b], b)
        pltpu.async_copy(bufs[b], dst(b), sems[b])

    def loop_body(j, carry):
        for b in range(2):
            ci = 2 * j + b
            pltpu.make_async_copy(bufs[b], dst(b), sems[b]).wait()
            fill(bufs[b], ci)
            pltpu.async_copy(bufs[b], dst(ci), sems[b])
        return carry

    lax.fori_loop(1, _NCHUNK // 2, loop_body, 0)

    for b in range(2):
        pltpu.make_async_copy(bufs[b], dst(b), sems[b]).wait()


def kernel(x, W):
    mesh = plsc.VectorSubcoreMesh(core_axis_name="c", subcore_axis_name="s")
    k = pl.kernel(
        _body,
        out_type=jax.ShapeDtypeStruct((_BATCH * _PERIOD * _HID,), jnp.float32),
        mesh=mesh,
        scratch_types=[
            pltpu.VMEM((_ROWS * _HID,), jnp.float32),
            pltpu.VMEM((_CH_OUT,), jnp.float32),
            pltpu.VMEM((_CH_OUT,), jnp.float32),
            pltpu.SemaphoreType.DMA,
            pltpu.SemaphoreType.DMA,
        ],
    )
    return k(W.reshape(-1)).reshape(_BATCH, _PERIOD, _HID)


# speedup vs baseline: 13.2327x; 13.2327x over previous
import jax
import jax.numpy as jnp
from jax import lax
from jax.experimental import pallas as pl
from jax.experimental.pallas import tpu as pltpu
from jax.experimental.pallas import tpu_sc as plsc

_BATCH = 16384
_PERIOD = 50
_HID = 16

_NC = 2
_NS = 16
_NW = _NC * _NS
_N = _BATCH * _HID
_SLICE = _N // _NW


def _body(w_hbm, out_hbm, w_v, sem):
    wid = lax.axis_index("s") * _NC + lax.axis_index("c")
    off = wid * _SLICE
    pltpu.sync_copy(w_hbm.at[pl.ds(off, _SLICE)], w_v)
    copies = [
        pltpu.async_copy(w_v, out_hbm.at[pl.ds(p * _N + off, _SLICE)], sem)
        for p in range(_PERIOD)
    ]
    for c in copies:
        c.wait()


def kernel(x, W):
    mesh = plsc.VectorSubcoreMesh(core_axis_name="c", subcore_axis_name="s")
    k = pl.kernel(
        _body,
        out_type=jax.ShapeDtypeStruct((_PERIOD * _N,), jnp.float32),
        mesh=mesh,
        scratch_types=[
            pltpu.VMEM((_SLICE,), jnp.float32),
            pltpu.SemaphoreType.DMA,
        ],
    )
    w_flat = W.reshape(128, 128, 2, 8).transpose(2, 0, 3, 1).reshape(-1)
    flat = k(w_flat)
    return (
        flat.reshape(_PERIOD, 2, 128, 8, 128)
        .transpose(2, 4, 0, 1, 3)
        .reshape(_BATCH, _PERIOD, _HID)
    )
